# baseline (speedup 1.0000x reference)
import jax
import jax.numpy as jnp
from jax import lax
from jax.experimental import pallas as pl
from jax.experimental.pallas import tpu as pltpu

M = 32768
D_MODEL = 768
NUM_EXPERTS = 64
CLAMP_MIN = -10000.0
CLAMP_MAX = 10000.0

BLOCK_M = 4096

_DIMS = (((1,), (1,)), ((), ()))


def _router_kernel(w_ref, b_ref, x_ref, out_ref, out2_ref):
    xb = x_ref[...].astype(jnp.bfloat16)
    wb = w_ref[...].astype(jnp.bfloat16)
    raw = lax.dot_general(wb, xb, _DIMS, preferred_element_type=jnp.float32)
    raw = raw + b_ref[...].T
    res = jnp.clip(raw, CLAMP_MIN, CLAMP_MAX)
    out_ref[...] = res
    out2_ref[...] = res


def kernel(x, W, expert_bias):
    bias = expert_bias.reshape(1, NUM_EXPERTS)
    grid = (M // BLOCK_M,)
    out_block = pl.BlockSpec((NUM_EXPERTS, BLOCK_M), lambda i: (0, i))
    out_shape = jax.ShapeDtypeStruct((NUM_EXPERTS, M), jnp.float32)
    o1, o2 = pl.pallas_call(
        _router_kernel,
        grid=grid,
        in_specs=[
            pl.BlockSpec((NUM_EXPERTS, D_MODEL), lambda i: (0, 0)),
            pl.BlockSpec((1, NUM_EXPERTS), lambda i: (0, 0)),
            pl.BlockSpec((BLOCK_M, D_MODEL), lambda i: (i, 0)),
        ],
        out_specs=(out_block, out_block),
        out_shape=(out_shape, out_shape),
        compiler_params=pltpu.CompilerParams(
            dimension_semantics=("parallel",),
        ),
    )(W, bias, x)
    return (o1.T, o2.T)


# speedup vs baseline: 1.0063x; 1.0063x over previous
import jax
import jax.numpy as jnp
from jax import lax
from jax.experimental import pallas as pl
from jax.experimental.pallas import tpu as pltpu

M = 32768
D_MODEL = 768
NUM_EXPERTS = 64
CLAMP_MIN = -10000.0
CLAMP_MAX = 10000.0

BLOCK_M = 4096

_DIMS = (((1,), (1,)), ((), ()))


def _router_kernel(w_ref, b_ref, xa_ref, xb_ref, out_ref, out2_ref):
    wb = w_ref[...].astype(jnp.bfloat16)
    b = b_ref[...].T
    ra = lax.dot_general(wb, xa_ref[...].astype(jnp.bfloat16), _DIMS,
                         preferred_element_type=jnp.float32)
    rb = lax.dot_general(wb, xb_ref[...].astype(jnp.bfloat16), _DIMS,
                         preferred_element_type=jnp.float32)
    resa = jnp.clip(ra + b, CLAMP_MIN, CLAMP_MAX)
    resb = jnp.clip(rb + b, CLAMP_MIN, CLAMP_MAX)
    out_ref[:, :HALF_M] = resa
    out_ref[:, HALF_M:] = resb
    out2_ref[:, :HALF_M] = resa
    out2_ref[:, HALF_M:] = resb


HALF_M = BLOCK_M // 2


def kernel(x, W, expert_bias):
    bias = expert_bias.reshape(1, NUM_EXPERTS)
    grid = (M // BLOCK_M,)
    out_block = pl.BlockSpec((NUM_EXPERTS, BLOCK_M), lambda i: (0, i))
    out_shape = jax.ShapeDtypeStruct((NUM_EXPERTS, M), jnp.float32)
    o1, o2 = pl.pallas_call(
        _router_kernel,
        grid=grid,
        in_specs=[
            pl.BlockSpec((NUM_EXPERTS, D_MODEL), lambda i: (0, 0)),
            pl.BlockSpec((1, NUM_EXPERTS), lambda i: (0, 0)),
            pl.BlockSpec((HALF_M, D_MODEL), lambda i: (2 * i, 0)),
            pl.BlockSpec((HALF_M, D_MODEL), lambda i: (2 * i + 1, 0)),
        ],
        out_specs=(out_block, out_block),
        out_shape=(out_shape, out_shape),
        compiler_params=pltpu.CompilerParams(
            dimension_semantics=("parallel",),
        ),
    )(W, bias, x, x)
    return (o1.T, o2.T)
